# baseline (speedup 1.0000x reference)
import functools

import jax
import jax.numpy as jnp
from jax import lax
from jax.experimental import pallas as pl
from jax.experimental.pallas import tpu as pltpu
from jax.experimental.pallas import tpu_sc as plsc

VOCAB = 100000
EMBED = 64
BATCH = 1024
CTX = 50

LANES = 128
NC = 2
NS = 16
NW = NC * NS
NB = BATCH // NW
WPS = LANES // NB


TB = 2048
TGRID = (VOCAB + TB - 1) // TB


def _tr_body(t_ref, o_ref):
    o_ref[:, 0:EMBED] = jnp.transpose(t_ref[...], (1, 0))


def _stage_table(tableT):
    return pl.pallas_call(
        _tr_body,
        grid=(TGRID,),
        in_specs=[pl.BlockSpec((EMBED, TB), lambda i: (0, i))],
        out_specs=pl.BlockSpec((TB, LANES), lambda i: (i, 0)),
        out_shape=jax.ShapeDtypeStruct((VOCAB, LANES), jnp.float32),
        compiler_params=pltpu.CompilerParams(
            dimension_semantics=("arbitrary",),
        ),
    )(tableT)


_sc_mesh = plsc.VectorSubcoreMesh(core_axis_name="c", subcore_axis_name="s")


@functools.partial(
    pl.kernel,
    mesh=_sc_mesh,
    out_type=jax.ShapeDtypeStruct((BATCH, EMBED), jnp.float32),
    scratch_types=[
        pltpu.VMEM((CTX, LANES), jnp.int32),
        pltpu.VMEM((2, NB, LANES), jnp.float32),
        pltpu.VMEM((NB, EMBED), jnp.float32),
        pltpu.SemaphoreType.DMA,
        pltpu.SemaphoreType.DMA,
    ],
)
def _pool_sc(idx_hbm, table_hbm, out_hbm, idx_v, rows_v, acc_v, sem0, sem1):
    wid = lax.axis_index("s") * NC + lax.axis_index("c")
    base_b = wid * NB
    slab = wid // WPS
    sub = wid % WPS

    pltpu.sync_copy(idx_hbm.at[:, pl.ds(slab * LANES, LANES)], idx_v)

    sems = (sem0, sem1)

    def fire(c, buf):
        pltpu.async_copy(
            table_hbm.at[idx_v.at[c, pl.ds(sub * NB, NB)]],
            rows_v.at[buf],
            sems[0] if buf == 0 else sems[1],
        )

    def drain(buf):
        pltpu.make_async_copy(
            table_hbm.at[idx_v.at[0, pl.ds(0, NB)]],
            rows_v.at[0],
            sems[0] if buf == 0 else sems[1],
        ).wait()

    zero = jnp.zeros((16,), jnp.float32)

    def z_body(b, carry):
        for g in range(EMBED // 16):
            acc_v[b, pl.ds(g * 16, 16)] = zero
        return carry

    lax.fori_loop(0, NB, z_body, 0)

    fire(0, 0)

    def step(c, carry):
        parity = lax.rem(c, 2)

        @pl.when(parity == 0)
        def _():
            drain(0)

            @pl.when(c + 1 < CTX)
            def _():
                fire(c + 1, 1)

        @pl.when(parity == 1)
        def _():
            drain(1)

            @pl.when(c + 1 < CTX)
            def _():
                fire(c + 1, 0)

        def acc_body(b, inner):
            for k in range(2):

                @pl.when(parity == k)
                def _(k=k):
                    for g in range(EMBED // 16):
                        plsc.addupdate(
                            acc_v.at[b, pl.ds(g * 16, 16)],
                            rows_v[k, b, pl.ds(g * 16, 16)],
                        )

            return inner

        lax.fori_loop(0, NB, acc_body, 0)
        return carry

    lax.fori_loop(0, CTX, step, 0)

    scale = jnp.float32(1.0 / CTX)

    def s_body(b, carry):
        for g in range(EMBED // 16):
            acc_v[b, pl.ds(g * 16, 16)] = acc_v[b, pl.ds(g * 16, 16)] * scale
        return carry

    lax.fori_loop(0, NB, s_body, 0)

    pltpu.sync_copy(acc_v, out_hbm.at[pl.ds(base_b, NB)])


VB = 4096
VGRID = (VOCAB + VB - 1) // VB


def _mm_body(x_ref, w_ref, b_ref, o_ref):
    acc = lax.dot_general(
        w_ref[...].astype(jnp.bfloat16),
        x_ref[...].astype(jnp.bfloat16),
        dimension_numbers=(((0,), (1,)), ((), ())),
        preferred_element_type=jnp.float32,
    )
    o_ref[...] = acc + jnp.transpose(b_ref[...], (1, 0))


def _dense_tc(x, W, b2d):
    out_t = pl.pallas_call(
        _mm_body,
        grid=(VGRID,),
        in_specs=[
            pl.BlockSpec((BATCH, EMBED), lambda i: (0, 0)),
            pl.BlockSpec((EMBED, VB), lambda i: (0, i)),
            pl.BlockSpec((1, VB), lambda i: (0, i)),
        ],
        out_specs=pl.BlockSpec((VB, BATCH), lambda i: (i, 0)),
        out_shape=jax.ShapeDtypeStruct((VOCAB, BATCH), jnp.float32),
        compiler_params=pltpu.CompilerParams(
            dimension_semantics=("arbitrary",),
        ),
    )(x, W, b2d)
    return out_t.T


def kernel(inputs, emb_table, W, b):
    staged = _stage_table(emb_table.T)
    pooled = _pool_sc(inputs.astype(jnp.int32).T, staged)
    return _dense_tc(pooled, W, b.reshape(1, VOCAB))
